# baseline (speedup 1.0000x reference)
import functools

import jax
import jax.numpy as jnp
from jax import lax
from jax.experimental import pallas as pl
from jax.experimental.pallas import tpu as pltpu
from jax.experimental.pallas import tpu_sc as plsc

GAMMA = 0.1
BPARAM = 10.0
TAO = 1.0
BETA = 1.0
LAMBDA_ = 0.1
EPS = 1e-6
C = 16
P = 512
D = 64
BATCH = 1024


def _assign_tc_kernel(labels_ref, feat_ref, protos_in, counts_in,
                      protos_out, counts_out):
    protos_out[...] = protos_in[...]
    counts_out[...] = counts_in[...]
    iota = jax.lax.broadcasted_iota(jnp.int32, (P, 1), 0)

    def body(i, carry):
        lab = labels_ref[i]
        frow = feat_ref[pl.ds(i, 1), :]
        cp = protos_out[lab]
        diff = frow - cp + EPS
        sq = jnp.sum(diff * diff, axis=1, keepdims=True)
        sq = jnp.maximum(sq, 1e-12)
        minval = jnp.min(sq)
        idx = jnp.min(jnp.where(sq == minval, iota, P))
        onehot = iota == idx
        cnt_col = counts_out[lab]
        cval = jnp.sum(jnp.where(onehot, cnt_col, 0.0))
        psel = jnp.sum(jnp.where(onehot, cp, 0.0), axis=0, keepdims=True)
        newp = (psel * cval + frow) / (cval + 1.0)
        protos_out[lab] = jnp.where(onehot, newp, cp)
        counts_out[lab] = cnt_col + jnp.where(onehot, 1.0, 0.0)
        return carry

    jax.lax.fori_loop(0, BATCH, body, 0)


L = 16


def _assign_sc(features, labels, protos_flat, counts):
    mesh = plsc.VectorSubcoreMesh(core_axis_name="c", subcore_axis_name="s")

    @functools.partial(
        pl.kernel,
        mesh=mesh,
        out_type=jax.ShapeDtypeStruct((C, P * D), jnp.float32),
        compiler_params=pltpu.CompilerParams(needs_layout_passes=False),
        scratch_types=[
            pltpu.VMEM((BATCH,), jnp.int32),
            pltpu.VMEM((BATCH,), jnp.int32),
            pltpu.VMEM((BATCH * D,), jnp.float32),
            pltpu.VMEM((P * D,), jnp.float32),
            pltpu.VMEM((P,), jnp.float32),
        ],
    )
    def k(feat_hbm, labels_hbm, protos_hbm, counts_hbm, out_hbm,
          labels_v, myidx_v, featbuf, protos_v, counts_v):
        cid = lax.axis_index("c")
        sid = lax.axis_index("s")
        cls = cid * 8 + sid
        lane = lax.iota(jnp.int32, L)

        @pl.when(sid < 8)
        def _body():
            pltpu.sync_copy(labels_hbm, labels_v)
            pltpu.sync_copy(protos_hbm.at[cls], protos_v)
            pltpu.sync_copy(counts_hbm.at[cls], counts_v)
            pltpu.sync_copy(feat_hbm, featbuf)

            ones_i = jnp.full((L,), 1, jnp.int32)
            zeros_i = jnp.full((L,), 0, jnp.int32)

            def cbody(t, nacc):
                lv = labels_v[pl.ds(t * L, L)]
                mask = lv == cls
                cntv = plsc.all_reduce_population_count(mask)
                keys = jnp.where(mask, zeros_i, ones_i)
                _, sorted_idx = lax.sort((keys, lane + t * L), num_keys=1)
                plsc.store_scatter(myidx_v, [nacc + lane], sorted_idx,
                                   mask=lane < cntv)
                return nacc + cntv[0]

            n = lax.fori_loop(0, BATCH // L, cbody, 0)

            big = jnp.float32(3.0e38)

            def sample_body(i, _):
                smpv = plsc.load_gather(myidx_v, [jnp.full((L,), i, jnp.int32)])
                f_eps = [plsc.load_gather(featbuf, [smpv * D + q * L + lane])
                         + EPS for q in range(D // L)]

                def grp(g, carry):
                    mv, mi = carry
                    acc = jnp.zeros((L,), jnp.float32)
                    base = g * (L * D) + lane * D
                    for d in range(D):
                        fd = f_eps[d // L][d % L]
                        pv = plsc.load_gather(protos_v, [base + d])
                        t = fd - pv
                        acc = acc + t * t
                    acc = jnp.maximum(acc, 1e-12)
                    jidx = g * L + lane
                    lt = acc < mv
                    return (jnp.where(lt, acc, mv), jnp.where(lt, jidx, mi))

                minval, minidx = lax.fori_loop(
                    0, P // L, grp,
                    (jnp.full((L,), big), jnp.zeros((L,), jnp.int32)))
                sv = lax.sort(minval)
                gmin = sv[0]
                cand = jnp.where(minval == gmin, minidx,
                                 jnp.full((L,), P, jnp.int32))
                j = lax.sort(cand)[0]

                jsplat = jnp.full((L,), j, jnp.int32)
                cvec = plsc.load_gather(counts_v, [jsplat])
                newcv = cvec + 1.0
                for q in range(D // L):
                    idxs = j * D + q * L + lane
                    pq = plsc.load_gather(protos_v, [idxs])
                    fq = plsc.load_gather(featbuf, [smpv * D + q * L + lane])
                    plsc.store_scatter(protos_v, [idxs],
                                       (pq * cvec + fq) / newcv)
                plsc.store_scatter(counts_v, [jsplat], newcv, mask=lane == 0)
                return 0

            lax.fori_loop(0, n, sample_body, 0)
            pltpu.sync_copy(protos_v, out_hbm.at[cls])

    return k(features, labels, protos_flat, counts)


def _loss_tc_kernel(labels_ref, feat_ref, protos_ref,
                    out_ref, one_acc, num_acc, pw_acc):
    c = pl.program_id(0)

    @pl.when(c == 0)
    def _init():
        one_acc[...] = jnp.zeros_like(one_acc)
        num_acc[...] = jnp.zeros_like(num_acc)
        pw_acc[...] = jnp.zeros_like(pw_acc)

    pb = protos_ref[0]
    feats = feat_ref[...]
    dn = (((1,), (1,)), ((), ()))
    xy = jax.lax.dot_general(feats, pb, dn,
                             preferred_element_type=jnp.float32)
    ones_row = jnp.ones((1, D), jnp.float32)
    ynrow = jax.lax.dot_general(ones_row, pb * pb, dn,
                                preferred_element_type=jnp.float32)
    ysrow = jax.lax.dot_general(ones_row, pb, dn,
                                preferred_element_type=jnp.float32)
    xn = jnp.sum(feats * feats, axis=1, keepdims=True)
    xs = jnp.sum(feats, axis=1, keepdims=True)
    sq = xn + ynrow - 2.0 * xy + 2.0 * EPS * (xs - ysrow) + D * EPS * EPS
    sq = jnp.maximum(sq, 1e-12)
    expterm = jnp.exp(-GAMMA * sq)
    pc = jnp.sum(expterm, axis=1, keepdims=True)
    lab = labels_ref[...]
    mask = lab == c
    one_acc[...] += pc
    num_acc[...] += jnp.where(mask, pc, 0.0)
    dmin = jnp.sqrt(jnp.min(sq, axis=1, keepdims=True))
    sign = jnp.where(mask, 1.0, -1.0)
    z = BPARAM - (TAO - dmin) * sign
    soft = jnp.log(1.0 + jnp.exp(BETA * jnp.minimum(z, 10.0))) / BETA
    pw_acc[...] += jnp.where(z > 10.0, z, soft)

    @pl.when(c == C - 1)
    def _fin():
        one = one_acc[...]
        num = num_acc[...]
        safe = jnp.where(one > 0.0, one, 1.0)
        prob = jnp.where(one > 0.0, 1e-6 + num / safe, 1e-6 + one)
        dce = jnp.sum(-jnp.log(prob))
        pw = jnp.sum(pw_acc[...])
        out_ref[...] = jnp.reshape(dce + LAMBDA_ * pw, (1, 1))


def _assign(features, labels, prototypes, counts3, interpret=False):
    return pl.pallas_call(
        _assign_tc_kernel,
        out_shape=[
            jax.ShapeDtypeStruct((C, P, D), jnp.float32),
            jax.ShapeDtypeStruct((C, P, 1), jnp.float32),
        ],
        in_specs=[
            pl.BlockSpec(memory_space=pltpu.SMEM),
            pl.BlockSpec(memory_space=pltpu.VMEM),
            pl.BlockSpec(memory_space=pltpu.VMEM),
            pl.BlockSpec(memory_space=pltpu.VMEM),
        ],
        out_specs=[
            pl.BlockSpec(memory_space=pltpu.VMEM),
            pl.BlockSpec(memory_space=pltpu.VMEM),
        ],
        interpret=interpret,
    )(labels, features, prototypes, counts3)


def _loss(labels2d, features, protos, interpret=False):
    return pl.pallas_call(
        _loss_tc_kernel,
        grid=(C,),
        out_shape=jax.ShapeDtypeStruct((1, 1), jnp.float32),
        in_specs=[
            pl.BlockSpec((BATCH, 1), lambda c: (0, 0)),
            pl.BlockSpec((BATCH, D), lambda c: (0, 0)),
            pl.BlockSpec((1, P, D), lambda c: (c, 0, 0)),
        ],
        out_specs=pl.BlockSpec((1, 1), lambda c: (0, 0)),
        scratch_shapes=[
            pltpu.VMEM((BATCH, 1), jnp.float32),
            pltpu.VMEM((BATCH, 1), jnp.float32),
            pltpu.VMEM((BATCH, 1), jnp.float32),
        ],
        interpret=interpret,
    )(labels2d, features, protos)


def kernel(features, labels, prototypes, counts):
    labels = labels.astype(jnp.int32)
    protos_up_flat = _assign_sc(features.reshape(BATCH * D), labels,
                                prototypes.reshape(C, P * D), counts)
    out = _loss(labels[:, None], features, protos_up_flat.reshape(C, P, D))
    return out[0, 0]


# speedup vs baseline: 7.4091x; 7.4091x over previous
import functools

import jax
import jax.numpy as jnp
from jax import lax
from jax.experimental import pallas as pl
from jax.experimental.pallas import tpu as pltpu
from jax.experimental.pallas import tpu_sc as plsc

GAMMA = 0.1
BPARAM = 10.0
TAO = 1.0
BETA = 1.0
LAMBDA_ = 0.1
EPS = 1e-6
C = 16
P = 512
D = 64
BATCH = 1024


def _assign_tc_kernel(labels_ref, feat_ref, protos_in, counts_in,
                      protos_out, counts_out):
    protos_out[...] = protos_in[...]
    counts_out[...] = counts_in[...]
    iota = jax.lax.broadcasted_iota(jnp.int32, (P, 1), 0)

    def body(i, carry):
        lab = labels_ref[i]
        frow = feat_ref[pl.ds(i, 1), :]
        cp = protos_out[lab]
        diff = frow - cp + EPS
        sq = jnp.sum(diff * diff, axis=1, keepdims=True)
        sq = jnp.maximum(sq, 1e-12)
        minval = jnp.min(sq)
        idx = jnp.min(jnp.where(sq == minval, iota, P))
        onehot = iota == idx
        cnt_col = counts_out[lab]
        cval = jnp.sum(jnp.where(onehot, cnt_col, 0.0))
        psel = jnp.sum(jnp.where(onehot, cp, 0.0), axis=0, keepdims=True)
        newp = (psel * cval + frow) / (cval + 1.0)
        protos_out[lab] = jnp.where(onehot, newp, cp)
        counts_out[lab] = cnt_col + jnp.where(onehot, 1.0, 0.0)
        return carry

    jax.lax.fori_loop(0, BATCH, body, 0)


L = 16
CP = C * P


def _scores0_tc_kernel(feat_ref, protos_ref, out_ref):
    pb = protos_ref[0]
    feats = feat_ref[...]
    dn = (((1,), (1,)), ((), ()))
    xy = jax.lax.dot_general(feats, pb, dn,
                             preferred_element_type=jnp.float32)
    ones_row = jnp.ones((1, D), jnp.float32)
    ynrow = jax.lax.dot_general(ones_row, pb * pb, dn,
                                preferred_element_type=jnp.float32)
    ysrow = jax.lax.dot_general(ones_row, pb, dn,
                                preferred_element_type=jnp.float32)
    out_ref[...] = ynrow - 2.0 * xy - 2.0 * EPS * ysrow


def _scores0(features, protos):
    return pl.pallas_call(
        _scores0_tc_kernel,
        grid=(C,),
        out_shape=jax.ShapeDtypeStruct((BATCH, CP), jnp.float32),
        in_specs=[
            pl.BlockSpec((BATCH, D), lambda c: (0, 0)),
            pl.BlockSpec((1, P, D), lambda c: (c, 0, 0)),
        ],
        out_specs=pl.BlockSpec((BATCH, P), lambda c: (0, c)),
    )(features, protos)


def _assign_sc(feat_flat, labels, protos_flat, counts, scores0_flat):
    mesh = plsc.VectorSubcoreMesh(core_axis_name="c", subcore_axis_name="s")

    @functools.partial(
        pl.kernel,
        mesh=mesh,
        out_type=jax.ShapeDtypeStruct((C, P * D), jnp.float32),
        compiler_params=pltpu.CompilerParams(needs_layout_passes=False),
        scratch_types=[
            pltpu.VMEM((BATCH,), jnp.int32),
            pltpu.VMEM((BATCH,), jnp.int32),
            pltpu.VMEM((P * D,), jnp.float32),
            pltpu.VMEM((P * D,), jnp.float32),
            pltpu.VMEM((P,), jnp.float32),
            pltpu.VMEM((2 * P,), jnp.float32),
            pltpu.VMEM((2 * D,), jnp.float32),
            pltpu.VMEM((P,), jnp.float32),
            pltpu.VMEM((P,), jnp.int32),
            pltpu.VMEM((P,), jnp.int32),
            pltpu.SemaphoreType.DMA,
            pltpu.SemaphoreType.DMA,
        ],
    )
    def k(feat_hbm, labels_hbm, protos_hbm, counts_hbm, scores0_hbm, out_hbm,
          labels_v, myidx_v, protos_v, protosT_v, counts_v,
          scorebuf, frow, ynys_v, dlist, dflag, sem_s, sem_f):
        cid = lax.axis_index("c")
        sid = lax.axis_index("s")
        cls = cid * 8 + sid
        lane = lax.iota(jnp.int32, L)

        @pl.when(sid < 8)
        def _body():
            pltpu.sync_copy(labels_hbm, labels_v)
            pltpu.sync_copy(protos_hbm.at[cls], protos_v)
            pltpu.sync_copy(counts_hbm.at[cls], counts_v)

            ones_i = jnp.full((L,), 1, jnp.int32)
            zeros_i = jnp.full((L,), 0, jnp.int32)
            zeros_f = jnp.zeros((L,), jnp.float32)

            def cbody(t, nacc):
                lv = labels_v[pl.ds(t * L, L)]
                mask = lv == cls
                cntv = plsc.all_reduce_population_count(mask)
                keys = jnp.where(mask, zeros_i, ones_i)
                _, sorted_idx = lax.sort((keys, lane + t * L), num_keys=1)
                plsc.store_scatter(myidx_v, [nacc + lane], sorted_idx,
                                   mask=lane < cntv)
                return nacc + cntv[0]

            n = lax.fori_loop(0, BATCH // L, cbody, 0)

            def ibody(t, _):
                dflag[pl.ds(t * L, L)] = zeros_i
                dlist[pl.ds(t * L, L)] = zeros_i
                return 0

            lax.fori_loop(0, P // L, ibody, 0)

            big = jnp.float32(3.0e38)

            @pl.when(n > 0)
            def _chain():
                smp0 = plsc.load_gather(myidx_v, [zeros_i])[0]
                pltpu.async_copy(
                    scores0_hbm.at[pl.ds(smp0 * CP + cls * P, P)],
                    scorebuf.at[pl.ds(0, P)], sem_s)
                pltpu.async_copy(feat_hbm.at[pl.ds(smp0 * D, D)],
                                 frow.at[pl.ds(0, D)], sem_f)

                def sample_body(i, ndirty):
                    par = lax.rem(i, 2)
                    ps = par * P
                    pf = par * D
                    pltpu.make_async_copy(
                        scores0_hbm.at[pl.ds(0, P)],
                        scorebuf.at[pl.ds(ps, P)], sem_s).wait()
                    pltpu.make_async_copy(
                        feat_hbm.at[pl.ds(0, D)],
                        frow.at[pl.ds(pf, D)], sem_f).wait()

                    @pl.when(i + 1 < n)
                    def _pf():
                        nx = plsc.load_gather(
                            myidx_v, [jnp.full((L,), i + 1, jnp.int32)])[0]
                        npar = lax.rem(i + 1, 2)
                        pltpu.async_copy(
                            scores0_hbm.at[pl.ds(nx * CP + cls * P, P)],
                            scorebuf.at[pl.ds(npar * P, P)], sem_s)
                        pltpu.async_copy(feat_hbm.at[pl.ds(nx * D, D)],
                                         frow.at[pl.ds(npar * D, D)], sem_f)

                    fr = [frow[pl.ds(pf + q * L, L)] for q in range(D // L)]

                    def corr(t, _):
                        jvec = dlist[pl.ds(t * L, L)]
                        valid = (t * L + lane) < ndirty
                        ynysv = plsc.load_gather(ynys_v, [jvec])
                        acc = zeros_f
                        for d in range(D):
                            pd = plsc.load_gather(protosT_v, [d * P + jvec])
                            acc = acc + fr[d // L][d % L] * pd
                        plsc.store_scatter(scorebuf, [ps + jvec],
                                           ynysv - 2.0 * acc, mask=valid)
                        return 0

                    lax.fori_loop(0, (ndirty + L - 1) // L, corr, 0)

                    def grp(g, carry):
                        mv, mi = carry
                        sv = scorebuf[pl.ds(ps + g * L, L)]
                        jidx = g * L + lane
                        lt = sv < mv
                        return (jnp.where(lt, sv, mv),
                                jnp.where(lt, jidx, mi))

                    minval, minidx = lax.fori_loop(
                        0, P // L, grp,
                        (jnp.full((L,), big), jnp.zeros((L,), jnp.int32)))
                    gmin = lax.sort(minval)[0]
                    cand = jnp.where(minval == gmin, minidx,
                                     jnp.full((L,), P, jnp.int32))
                    j = lax.sort(cand)[0]

                    jsplat = jnp.full((L,), j, jnp.int32)
                    cvec = plsc.load_gather(counts_v, [jsplat])
                    newcv = cvec + 1.0
                    accy = zeros_f
                    for q in range(D // L):
                        off = j * D + q * L
                        pq = protos_v[pl.ds(off, L)]
                        npq = (pq * cvec + fr[q]) / newcv
                        protos_v[pl.ds(off, L)] = npq
                        plsc.store_scatter(protosT_v,
                                           [(q * L + lane) * P + j], npq)
                        accy = accy + npq * npq - (2.0 * EPS) * npq
                    s = accy[0]
                    for r in range(1, L):
                        s = s + accy[r]
                    plsc.store_scatter(ynys_v, [jsplat],
                                       jnp.full((L,), s, jnp.float32),
                                       mask=lane == 0)
                    plsc.store_scatter(counts_v, [jsplat], newcv,
                                       mask=lane == 0)

                    flagv = plsc.load_gather(dflag, [jsplat])
                    newmask = (lane == 0) & (flagv == 0)
                    plsc.store_scatter(
                        dlist, [jnp.full((L,), ndirty, jnp.int32)], jsplat,
                        mask=newmask)
                    plsc.store_scatter(dflag, [jsplat], ones_i, mask=newmask)
                    return ndirty + 1 - flagv[0]

                lax.fori_loop(0, n, sample_body, 0)

            pltpu.sync_copy(protos_v, out_hbm.at[cls])

    return k(feat_flat, labels, protos_flat, counts, scores0_flat)


def _loss_tc_kernel(labels_ref, feat_ref, protos_ref,
                    out_ref, one_acc, num_acc, pw_acc):
    c = pl.program_id(0)

    @pl.when(c == 0)
    def _init():
        one_acc[...] = jnp.zeros_like(one_acc)
        num_acc[...] = jnp.zeros_like(num_acc)
        pw_acc[...] = jnp.zeros_like(pw_acc)

    pb = protos_ref[0]
    feats = feat_ref[...]
    dn = (((1,), (1,)), ((), ()))
    xy = jax.lax.dot_general(feats, pb, dn,
                             preferred_element_type=jnp.float32)
    ones_row = jnp.ones((1, D), jnp.float32)
    ynrow = jax.lax.dot_general(ones_row, pb * pb, dn,
                                preferred_element_type=jnp.float32)
    ysrow = jax.lax.dot_general(ones_row, pb, dn,
                                preferred_element_type=jnp.float32)
    xn = jnp.sum(feats * feats, axis=1, keepdims=True)
    xs = jnp.sum(feats, axis=1, keepdims=True)
    sq = xn + ynrow - 2.0 * xy + 2.0 * EPS * (xs - ysrow) + D * EPS * EPS
    sq = jnp.maximum(sq, 1e-12)
    expterm = jnp.exp(-GAMMA * sq)
    pc = jnp.sum(expterm, axis=1, keepdims=True)
    lab = labels_ref[...]
    mask = lab == c
    one_acc[...] += pc
    num_acc[...] += jnp.where(mask, pc, 0.0)
    dmin = jnp.sqrt(jnp.min(sq, axis=1, keepdims=True))
    sign = jnp.where(mask, 1.0, -1.0)
    z = BPARAM - (TAO - dmin) * sign
    soft = jnp.log(1.0 + jnp.exp(BETA * jnp.minimum(z, 10.0))) / BETA
    pw_acc[...] += jnp.where(z > 10.0, z, soft)

    @pl.when(c == C - 1)
    def _fin():
        one = one_acc[...]
        num = num_acc[...]
        safe = jnp.where(one > 0.0, one, 1.0)
        prob = jnp.where(one > 0.0, 1e-6 + num / safe, 1e-6 + one)
        dce = jnp.sum(-jnp.log(prob))
        pw = jnp.sum(pw_acc[...])
        out_ref[...] = jnp.reshape(dce + LAMBDA_ * pw, (1, 1))


def _assign(features, labels, prototypes, counts3, interpret=False):
    return pl.pallas_call(
        _assign_tc_kernel,
        out_shape=[
            jax.ShapeDtypeStruct((C, P, D), jnp.float32),
            jax.ShapeDtypeStruct((C, P, 1), jnp.float32),
        ],
        in_specs=[
            pl.BlockSpec(memory_space=pltpu.SMEM),
            pl.BlockSpec(memory_space=pltpu.VMEM),
            pl.BlockSpec(memory_space=pltpu.VMEM),
            pl.BlockSpec(memory_space=pltpu.VMEM),
        ],
        out_specs=[
            pl.BlockSpec(memory_space=pltpu.VMEM),
            pl.BlockSpec(memory_space=pltpu.VMEM),
        ],
        interpret=interpret,
    )(labels, features, prototypes, counts3)


def _loss(labels2d, features, protos, interpret=False):
    return pl.pallas_call(
        _loss_tc_kernel,
        grid=(C,),
        out_shape=jax.ShapeDtypeStruct((1, 1), jnp.float32),
        in_specs=[
            pl.BlockSpec((BATCH, 1), lambda c: (0, 0)),
            pl.BlockSpec((BATCH, D), lambda c: (0, 0)),
            pl.BlockSpec((1, P, D), lambda c: (c, 0, 0)),
        ],
        out_specs=pl.BlockSpec((1, 1), lambda c: (0, 0)),
        scratch_shapes=[
            pltpu.VMEM((BATCH, 1), jnp.float32),
            pltpu.VMEM((BATCH, 1), jnp.float32),
            pltpu.VMEM((BATCH, 1), jnp.float32),
        ],
        interpret=interpret,
    )(labels2d, features, protos)


def kernel(features, labels, prototypes, counts):
    labels = labels.astype(jnp.int32)
    s0 = _scores0(features, prototypes)
    protos_up_flat = _assign_sc(features.reshape(BATCH * D), labels,
                                prototypes.reshape(C, P * D), counts,
                                s0.reshape(BATCH * CP))
    out = _loss(labels[:, None], features, protos_up_flat.reshape(C, P, D))
    return out[0, 0]


# speedup vs baseline: 8.8658x; 1.1966x over previous
import functools

import jax
import jax.numpy as jnp
from jax import lax
from jax.experimental import pallas as pl
from jax.experimental.pallas import tpu as pltpu
from jax.experimental.pallas import tpu_sc as plsc

GAMMA = 0.1
BPARAM = 10.0
TAO = 1.0
BETA = 1.0
LAMBDA_ = 0.1
EPS = 1e-6
C = 16
P = 512
D = 64
BATCH = 1024


def _assign_tc_kernel(labels_ref, feat_ref, protos_in, counts_in,
                      protos_out, counts_out):
    protos_out[...] = protos_in[...]
    counts_out[...] = counts_in[...]
    iota = jax.lax.broadcasted_iota(jnp.int32, (P, 1), 0)

    def body(i, carry):
        lab = labels_ref[i]
        frow = feat_ref[pl.ds(i, 1), :]
        cp = protos_out[lab]
        diff = frow - cp + EPS
        sq = jnp.sum(diff * diff, axis=1, keepdims=True)
        sq = jnp.maximum(sq, 1e-12)
        minval = jnp.min(sq)
        idx = jnp.min(jnp.where(sq == minval, iota, P))
        onehot = iota == idx
        cnt_col = counts_out[lab]
        cval = jnp.sum(jnp.where(onehot, cnt_col, 0.0))
        psel = jnp.sum(jnp.where(onehot, cp, 0.0), axis=0, keepdims=True)
        newp = (psel * cval + frow) / (cval + 1.0)
        protos_out[lab] = jnp.where(onehot, newp, cp)
        counts_out[lab] = cnt_col + jnp.where(onehot, 1.0, 0.0)
        return carry

    jax.lax.fori_loop(0, BATCH, body, 0)


L = 16
CP = C * P


def _scores0_tc_kernel(feat_ref, protos_ref, out_ref):
    pb = protos_ref[0]
    feats = feat_ref[...]
    dn = (((1,), (1,)), ((), ()))
    xy = jax.lax.dot_general(feats, pb, dn,
                             preferred_element_type=jnp.float32)
    ones_row = jnp.ones((1, D), jnp.float32)
    ynrow = jax.lax.dot_general(ones_row, pb * pb, dn,
                                preferred_element_type=jnp.float32)
    ysrow = jax.lax.dot_general(ones_row, pb, dn,
                                preferred_element_type=jnp.float32)
    out_ref[...] = (ynrow - 2.0 * xy - 2.0 * EPS * ysrow)[None, :, :]


def _scores0(features, protos):
    return pl.pallas_call(
        _scores0_tc_kernel,
        grid=(C,),
        out_shape=jax.ShapeDtypeStruct((C, BATCH, P), jnp.float32),
        in_specs=[
            pl.BlockSpec((BATCH, D), lambda c: (0, 0)),
            pl.BlockSpec((1, P, D), lambda c: (c, 0, 0)),
        ],
        out_specs=pl.BlockSpec((1, BATCH, P), lambda c: (c, 0, 0)),
    )(features, protos)


def _assign_sc(feat_flat, labels, protos_flat, counts, scores0_flat):
    mesh = plsc.VectorSubcoreMesh(core_axis_name="c", subcore_axis_name="s")

    @functools.partial(
        pl.kernel,
        mesh=mesh,
        out_type=jax.ShapeDtypeStruct((C, P, D), jnp.float32),
        compiler_params=pltpu.CompilerParams(needs_layout_passes=False),
        scratch_types=[
            pltpu.VMEM((BATCH,), jnp.int32),
            pltpu.VMEM((BATCH,), jnp.int32),
            pltpu.VMEM((P, D), jnp.float32),
            pltpu.VMEM((P * D,), jnp.float32),
            pltpu.VMEM((P,), jnp.float32),
            pltpu.VMEM((2, P), jnp.float32),
            pltpu.VMEM((2, D), jnp.float32),
            pltpu.VMEM((P,), jnp.float32),
            pltpu.VMEM((P,), jnp.int32),
            pltpu.VMEM((P,), jnp.int32),
            pltpu.SemaphoreType.DMA,
            pltpu.SemaphoreType.DMA,
        ],
    )
    def k(feat_hbm, labels_hbm, protos_hbm, counts_hbm, scores0_hbm, out_hbm,
          labels_v, myidx_v, protos_v, protosT_v, counts_v,
          scorebuf, frow, ynys_v, dlist, dflag, sem_s, sem_f):
        cid = lax.axis_index("c")
        sid = lax.axis_index("s")
        cls = cid * 8 + sid
        lane = lax.iota(jnp.int32, L)

        @pl.when(sid < 8)
        def _body():
            pltpu.sync_copy(labels_hbm, labels_v)
            pltpu.sync_copy(protos_hbm.at[cls], protos_v)
            pltpu.sync_copy(counts_hbm.at[cls], counts_v)

            ones_i = jnp.full((L,), 1, jnp.int32)
            zeros_i = jnp.full((L,), 0, jnp.int32)
            zeros_f = jnp.zeros((L,), jnp.float32)

            def cbody(t, nacc):
                lv = labels_v[pl.ds(t * L, L)]
                mask = lv == cls
                cntv = plsc.all_reduce_population_count(mask)
                keys = jnp.where(mask, zeros_i, ones_i)
                _, sorted_idx = lax.sort((keys, lane + t * L), num_keys=1)
                plsc.store_scatter(myidx_v, [nacc + lane], sorted_idx,
                                   mask=lane < cntv)
                return nacc + cntv[0]

            n = lax.fori_loop(0, BATCH // L, cbody, 0)

            def ibody(t, _):
                dflag[pl.ds(t * L, L)] = zeros_i
                dlist[pl.ds(t * L, L)] = zeros_i
                return 0

            lax.fori_loop(0, P // L, ibody, 0)

            big = jnp.float32(3.0e38)

            @pl.when(n > 0)
            def _chain():
                smp0 = plsc.load_gather(myidx_v, [zeros_i])[0]
                pltpu.async_copy(scores0_hbm.at[cls, smp0],
                                 scorebuf.at[0], sem_s)
                pltpu.async_copy(feat_hbm.at[smp0], frow.at[0], sem_f)

                def sample_body(i, ndirty):
                    par = lax.rem(i, 2)
                    pltpu.make_async_copy(scores0_hbm.at[0, 0],
                                          scorebuf.at[par], sem_s).wait()
                    pltpu.make_async_copy(feat_hbm.at[0],
                                          frow.at[par], sem_f).wait()

                    @pl.when(i + 1 < n)
                    def _pf():
                        nx = plsc.load_gather(
                            myidx_v, [jnp.full((L,), i + 1, jnp.int32)])[0]
                        npar = lax.rem(i + 1, 2)
                        pltpu.async_copy(scores0_hbm.at[cls, nx],
                                         scorebuf.at[npar], sem_s)
                        pltpu.async_copy(feat_hbm.at[nx], frow.at[npar],
                                         sem_f)

                    fr = [frow[par, pl.ds(q * L, L)] for q in range(D // L)]

                    def corr(t, _):
                        jvec = dlist[pl.ds(t * L, L)]
                        valid = (t * L + lane) < ndirty
                        ynysv = plsc.load_gather(ynys_v, [jvec])
                        acc = zeros_f
                        for d in range(D):
                            pd = plsc.load_gather(protosT_v, [d * P + jvec])
                            acc = acc + fr[d // L][d % L] * pd
                        plsc.store_scatter(
                            scorebuf, [jnp.full((L,), par, jnp.int32), jvec],
                            ynysv - 2.0 * acc, mask=valid)
                        return 0

                    lax.fori_loop(0, (ndirty + L - 1) // L, corr, 0)

                    def grp(g, carry):
                        mv, mi = carry
                        sv = scorebuf[par, pl.ds(g * L, L)]
                        jidx = g * L + lane
                        lt = sv < mv
                        return (jnp.where(lt, sv, mv),
                                jnp.where(lt, jidx, mi))

                    minval, minidx = lax.fori_loop(
                        0, P // L, grp,
                        (jnp.full((L,), big), jnp.zeros((L,), jnp.int32)))
                    gmin = lax.sort(minval)[0]
                    cand = jnp.where(minval == gmin, minidx,
                                     jnp.full((L,), P, jnp.int32))
                    j = lax.sort(cand)[0]

                    jsplat = jnp.full((L,), j, jnp.int32)
                    cvec = plsc.load_gather(counts_v, [jsplat])
                    newcv = cvec + 1.0
                    accy = zeros_f
                    for q in range(D // L):
                        pq = protos_v[j, pl.ds(q * L, L)]
                        npq = (pq * cvec + fr[q]) / newcv
                        protos_v[j, pl.ds(q * L, L)] = npq
                        plsc.store_scatter(protosT_v,
                                           [(q * L + lane) * P + j], npq)
                        accy = accy + npq * npq - (2.0 * EPS) * npq
                    s = accy[0]
                    for r in range(1, L):
                        s = s + accy[r]
                    plsc.store_scatter(ynys_v, [jsplat],
                                       jnp.full((L,), s, jnp.float32),
                                       mask=lane == 0)
                    plsc.store_scatter(counts_v, [jsplat], newcv,
                                       mask=lane == 0)

                    flagv = plsc.load_gather(dflag, [jsplat])
                    newmask = (lane == 0) & (flagv == 0)
                    plsc.store_scatter(
                        dlist, [jnp.full((L,), ndirty, jnp.int32)], jsplat,
                        mask=newmask)
                    plsc.store_scatter(dflag, [jsplat], ones_i, mask=newmask)
                    return ndirty + 1 - flagv[0]

                lax.fori_loop(0, n, sample_body, 0)

            pltpu.sync_copy(protos_v, out_hbm.at[cls])

    return k(feat_flat, labels, protos_flat, counts, scores0_flat)


def _loss_tc_kernel(labels_ref, feat_ref, protos_ref,
                    out_ref, one_acc, num_acc, pw_acc):
    c = pl.program_id(0)

    @pl.when(c == 0)
    def _init():
        one_acc[...] = jnp.zeros_like(one_acc)
        num_acc[...] = jnp.zeros_like(num_acc)
        pw_acc[...] = jnp.zeros_like(pw_acc)

    pb = protos_ref[0]
    feats = feat_ref[...]
    dn = (((1,), (1,)), ((), ()))
    xy = jax.lax.dot_general(feats, pb, dn,
                             preferred_element_type=jnp.float32)
    ones_row = jnp.ones((1, D), jnp.float32)
    ynrow = jax.lax.dot_general(ones_row, pb * pb, dn,
                                preferred_element_type=jnp.float32)
    ysrow = jax.lax.dot_general(ones_row, pb, dn,
                                preferred_element_type=jnp.float32)
    xn = jnp.sum(feats * feats, axis=1, keepdims=True)
    xs = jnp.sum(feats, axis=1, keepdims=True)
    sq = xn + ynrow - 2.0 * xy + 2.0 * EPS * (xs - ysrow) + D * EPS * EPS
    sq = jnp.maximum(sq, 1e-12)
    expterm = jnp.exp(-GAMMA * sq)
    pc = jnp.sum(expterm, axis=1, keepdims=True)
    lab = labels_ref[...]
    mask = lab == c
    one_acc[...] += pc
    num_acc[...] += jnp.where(mask, pc, 0.0)
    dmin = jnp.sqrt(jnp.min(sq, axis=1, keepdims=True))
    sign = jnp.where(mask, 1.0, -1.0)
    z = BPARAM - (TAO - dmin) * sign
    soft = jnp.log(1.0 + jnp.exp(BETA * jnp.minimum(z, 10.0))) / BETA
    pw_acc[...] += jnp.where(z > 10.0, z, soft)

    @pl.when(c == C - 1)
    def _fin():
        one = one_acc[...]
        num = num_acc[...]
        safe = jnp.where(one > 0.0, one, 1.0)
        prob = jnp.where(one > 0.0, 1e-6 + num / safe, 1e-6 + one)
        dce = jnp.sum(-jnp.log(prob))
        pw = jnp.sum(pw_acc[...])
        out_ref[...] = jnp.reshape(dce + LAMBDA_ * pw, (1, 1))


def _assign(features, labels, prototypes, counts3, interpret=False):
    return pl.pallas_call(
        _assign_tc_kernel,
        out_shape=[
            jax.ShapeDtypeStruct((C, P, D), jnp.float32),
            jax.ShapeDtypeStruct((C, P, 1), jnp.float32),
        ],
        in_specs=[
            pl.BlockSpec(memory_space=pltpu.SMEM),
            pl.BlockSpec(memory_space=pltpu.VMEM),
            pl.BlockSpec(memory_space=pltpu.VMEM),
            pl.BlockSpec(memory_space=pltpu.VMEM),
        ],
        out_specs=[
            pl.BlockSpec(memory_space=pltpu.VMEM),
            pl.BlockSpec(memory_space=pltpu.VMEM),
        ],
        interpret=interpret,
    )(labels, features, prototypes, counts3)


def _loss(labels2d, features, protos, interpret=False):
    return pl.pallas_call(
        _loss_tc_kernel,
        grid=(C,),
        out_shape=jax.ShapeDtypeStruct((1, 1), jnp.float32),
        in_specs=[
            pl.BlockSpec((BATCH, 1), lambda c: (0, 0)),
            pl.BlockSpec((BATCH, D), lambda c: (0, 0)),
            pl.BlockSpec((1, P, D), lambda c: (c, 0, 0)),
        ],
        out_specs=pl.BlockSpec((1, 1), lambda c: (0, 0)),
        scratch_shapes=[
            pltpu.VMEM((BATCH, 1), jnp.float32),
            pltpu.VMEM((BATCH, 1), jnp.float32),
            pltpu.VMEM((BATCH, 1), jnp.float32),
        ],
        interpret=interpret,
    )(labels2d, features, protos)


def kernel(features, labels, prototypes, counts):
    labels = labels.astype(jnp.int32)
    s0 = _scores0(features, prototypes)
    protos_up = _assign_sc(features, labels, prototypes, counts, s0)
    out = _loss(labels[:, None], features, protos_up)
    return out[0, 0]


# speedup vs baseline: 9.9643x; 1.1239x over previous
import functools

import jax
import jax.numpy as jnp
from jax import lax
from jax.experimental import pallas as pl
from jax.experimental.pallas import tpu as pltpu
from jax.experimental.pallas import tpu_sc as plsc

GAMMA = 0.1
BPARAM = 10.0
TAO = 1.0
BETA = 1.0
LAMBDA_ = 0.1
EPS = 1e-6
C = 16
P = 512
D = 64
BATCH = 1024


def _assign_tc_kernel(labels_ref, feat_ref, protos_in, counts_in,
                      protos_out, counts_out):
    protos_out[...] = protos_in[...]
    counts_out[...] = counts_in[...]
    iota = jax.lax.broadcasted_iota(jnp.int32, (P, 1), 0)

    def body(i, carry):
        lab = labels_ref[i]
        frow = feat_ref[pl.ds(i, 1), :]
        cp = protos_out[lab]
        diff = frow - cp + EPS
        sq = jnp.sum(diff * diff, axis=1, keepdims=True)
        sq = jnp.maximum(sq, 1e-12)
        minval = jnp.min(sq)
        idx = jnp.min(jnp.where(sq == minval, iota, P))
        onehot = iota == idx
        cnt_col = counts_out[lab]
        cval = jnp.sum(jnp.where(onehot, cnt_col, 0.0))
        psel = jnp.sum(jnp.where(onehot, cp, 0.0), axis=0, keepdims=True)
        newp = (psel * cval + frow) / (cval + 1.0)
        protos_out[lab] = jnp.where(onehot, newp, cp)
        counts_out[lab] = cnt_col + jnp.where(onehot, 1.0, 0.0)
        return carry

    jax.lax.fori_loop(0, BATCH, body, 0)


L = 16
CP = C * P


def _scores0_tc_kernel(feat_ref, protos_ref, out_ref):
    pb = protos_ref[0]
    feats = feat_ref[...]
    dn = (((1,), (1,)), ((), ()))
    xy = jax.lax.dot_general(feats, pb, dn,
                             preferred_element_type=jnp.float32)
    ones_row = jnp.ones((1, D), jnp.float32)
    ynrow = jax.lax.dot_general(ones_row, pb * pb, dn,
                                preferred_element_type=jnp.float32)
    ysrow = jax.lax.dot_general(ones_row, pb, dn,
                                preferred_element_type=jnp.float32)
    out_ref[...] = (ynrow - 2.0 * xy - 2.0 * EPS * ysrow)[None, :, :]


def _scores0(features, protos):
    return pl.pallas_call(
        _scores0_tc_kernel,
        grid=(C,),
        out_shape=jax.ShapeDtypeStruct((C, BATCH, P), jnp.float32),
        in_specs=[
            pl.BlockSpec((BATCH, D), lambda c: (0, 0)),
            pl.BlockSpec((1, P, D), lambda c: (c, 0, 0)),
        ],
        out_specs=pl.BlockSpec((1, BATCH, P), lambda c: (c, 0, 0)),
    )(features, protos)


def _assign_sc(feat_flat, labels, protos_flat, counts, scores0_flat):
    mesh = plsc.VectorSubcoreMesh(core_axis_name="c", subcore_axis_name="s")

    @functools.partial(
        pl.kernel,
        mesh=mesh,
        out_type=jax.ShapeDtypeStruct((C, P, D), jnp.float32),
        compiler_params=pltpu.CompilerParams(needs_layout_passes=False),
        scratch_types=[
            pltpu.VMEM((BATCH,), jnp.int32),
            pltpu.VMEM((BATCH,), jnp.int32),
            pltpu.VMEM((P, D), jnp.float32),
            pltpu.VMEM((P * D,), jnp.float32),
            pltpu.VMEM((P,), jnp.float32),
            pltpu.VMEM((2, P), jnp.float32),
            pltpu.VMEM((2, D), jnp.float32),
            pltpu.VMEM((P,), jnp.float32),
            pltpu.VMEM((P,), jnp.int32),
            pltpu.VMEM((P,), jnp.int32),
            pltpu.SemaphoreType.DMA,
            pltpu.SemaphoreType.DMA,
        ],
    )
    def k(feat_hbm, labels_hbm, protos_hbm, counts_hbm, scores0_hbm, out_hbm,
          labels_v, myidx_v, protos_v, protosT_v, counts_v,
          scorebuf, frow, ynys_v, dlist, dflag, sem_s, sem_f):
        cid = lax.axis_index("c")
        sid = lax.axis_index("s")
        cls = cid * 8 + sid
        lane = lax.iota(jnp.int32, L)

        @pl.when(sid < 8)
        def _body():
            pltpu.sync_copy(labels_hbm, labels_v)
            pltpu.sync_copy(protos_hbm.at[cls], protos_v)
            pltpu.sync_copy(counts_hbm.at[cls], counts_v)

            ones_i = jnp.full((L,), 1, jnp.int32)
            zeros_i = jnp.full((L,), 0, jnp.int32)
            zeros_f = jnp.zeros((L,), jnp.float32)

            def cbody(t, nacc):
                lv = labels_v[pl.ds(t * L, L)]
                mask = lv == cls
                cntv = plsc.all_reduce_population_count(mask)
                keys = jnp.where(mask, zeros_i, ones_i)
                _, sorted_idx = lax.sort((keys, lane + t * L), num_keys=1)
                plsc.store_scatter(myidx_v, [nacc + lane], sorted_idx,
                                   mask=lane < cntv)
                return nacc + cntv[0]

            n = lax.fori_loop(0, BATCH // L, cbody, 0)

            def ibody(t, _):
                dflag[pl.ds(t * L, L)] = zeros_i
                dlist[pl.ds(t * L, L)] = zeros_i
                return 0

            lax.fori_loop(0, P // L, ibody, 0)

            big = jnp.float32(3.0e38)

            @pl.when(n > 0)
            def _chain():
                smp0 = plsc.load_gather(myidx_v, [zeros_i])[0]
                pltpu.async_copy(scores0_hbm.at[cls, smp0],
                                 scorebuf.at[0], sem_s)
                pltpu.async_copy(feat_hbm.at[smp0], frow.at[0], sem_f)

                def sample_body(i, ndirty):
                    par = lax.rem(i, 2)
                    pltpu.make_async_copy(scores0_hbm.at[0, 0],
                                          scorebuf.at[par], sem_s).wait()
                    pltpu.make_async_copy(feat_hbm.at[0],
                                          frow.at[par], sem_f).wait()

                    @pl.when(i + 1 < n)
                    def _pf():
                        nx = plsc.load_gather(
                            myidx_v, [jnp.full((L,), i + 1, jnp.int32)])[0]
                        npar = lax.rem(i + 1, 2)
                        pltpu.async_copy(scores0_hbm.at[cls, nx],
                                         scorebuf.at[npar], sem_s)
                        pltpu.async_copy(feat_hbm.at[nx], frow.at[npar],
                                         sem_f)

                    fr = [frow[par, pl.ds(q * L, L)] for q in range(D // L)]

                    def corr(t, _):
                        jvec = dlist[pl.ds(t * L, L)]
                        valid = (t * L + lane) < ndirty
                        ynysv = plsc.load_gather(ynys_v, [jvec])
                        acc = zeros_f
                        for d in range(D):
                            pd = plsc.load_gather(protosT_v, [d * P + jvec])
                            acc = acc + fr[d // L][d % L] * pd
                        plsc.store_scatter(
                            scorebuf, [jnp.full((L,), par, jnp.int32), jvec],
                            ynysv - 2.0 * acc, mask=valid)
                        return 0

                    lax.fori_loop(0, (ndirty + L - 1) // L, corr, 0)

                    def grp(g4, carry):
                        mv, mi = carry
                        for u in range(4):
                            g = g4 * 4 + u
                            sv = scorebuf[par, pl.ds(g * L, L)]
                            jidx = g * L + lane
                            lt = sv < mv
                            mv = jnp.where(lt, sv, mv)
                            mi = jnp.where(lt, jidx, mi)
                        return (mv, mi)

                    minval, minidx = lax.fori_loop(
                        0, P // (L * 4), grp,
                        (jnp.full((L,), big), jnp.zeros((L,), jnp.int32)))
                    gmin = lax.sort(minval)[0]
                    cand = jnp.where(minval == gmin, minidx,
                                     jnp.full((L,), P, jnp.int32))
                    j = lax.sort(cand)[0]

                    jsplat = jnp.full((L,), j, jnp.int32)
                    cvec = plsc.load_gather(counts_v, [jsplat])
                    newcv = cvec + 1.0
                    accy = zeros_f
                    for q in range(D // L):
                        pq = protos_v[j, pl.ds(q * L, L)]
                        npq = (pq * cvec + fr[q]) / newcv
                        protos_v[j, pl.ds(q * L, L)] = npq
                        plsc.store_scatter(protosT_v,
                                           [(q * L + lane) * P + j], npq)
                        accy = accy + npq * npq - (2.0 * EPS) * npq
                    s = accy[0]
                    for r in range(1, L):
                        s = s + accy[r]
                    plsc.store_scatter(ynys_v, [jsplat],
                                       jnp.full((L,), s, jnp.float32),
                                       mask=lane == 0)
                    plsc.store_scatter(counts_v, [jsplat], newcv,
                                       mask=lane == 0)

                    flagv = plsc.load_gather(dflag, [jsplat])
                    newmask = (lane == 0) & (flagv == 0)
                    plsc.store_scatter(
                        dlist, [jnp.full((L,), ndirty, jnp.int32)], jsplat,
                        mask=newmask)
                    plsc.store_scatter(dflag, [jsplat], ones_i, mask=newmask)
                    return ndirty + 1 - flagv[0]

                lax.fori_loop(0, n, sample_body, 0)

            pltpu.sync_copy(protos_v, out_hbm.at[cls])

    return k(feat_flat, labels, protos_flat, counts, scores0_flat)


def _loss_tc_kernel(labels_ref, feat_ref, protos_ref,
                    out_ref, one_acc, num_acc, pw_acc):
    c = pl.program_id(0)

    @pl.when(c == 0)
    def _init():
        one_acc[...] = jnp.zeros_like(one_acc)
        num_acc[...] = jnp.zeros_like(num_acc)
        pw_acc[...] = jnp.zeros_like(pw_acc)

    pb = protos_ref[0]
    feats = feat_ref[...]
    dn = (((1,), (1,)), ((), ()))
    xy = jax.lax.dot_general(feats, pb, dn,
                             preferred_element_type=jnp.float32)
    ones_row = jnp.ones((1, D), jnp.float32)
    ynrow = jax.lax.dot_general(ones_row, pb * pb, dn,
                                preferred_element_type=jnp.float32)
    ysrow = jax.lax.dot_general(ones_row, pb, dn,
                                preferred_element_type=jnp.float32)
    xn = jnp.sum(feats * feats, axis=1, keepdims=True)
    xs = jnp.sum(feats, axis=1, keepdims=True)
    xrow = xn + 2.0 * EPS * xs + D * EPS * EPS
    yrow = ynrow - 2.0 * EPS * ysrow
    sq = (xrow + yrow) - 2.0 * xy
    sq = jnp.maximum(sq, 1e-12)
    expterm = jnp.exp(-GAMMA * sq)
    ones_col = jnp.ones((P, 1), jnp.float32)
    pc = jax.lax.dot_general(expterm, ones_col,
                             (((1,), (0,)), ((), ())),
                             preferred_element_type=jnp.float32)
    lab = labels_ref[...]
    mask = lab == c
    one_acc[...] += pc
    num_acc[...] += jnp.where(mask, pc, 0.0)
    dmin = jnp.sqrt(jnp.min(sq, axis=1, keepdims=True))
    sign = jnp.where(mask, 1.0, -1.0)
    z = BPARAM - (TAO - dmin) * sign
    soft = jnp.log(1.0 + jnp.exp(BETA * jnp.minimum(z, 10.0))) / BETA
    pw_acc[...] += jnp.where(z > 10.0, z, soft)

    @pl.when(c == C - 1)
    def _fin():
        one = one_acc[...]
        num = num_acc[...]
        safe = jnp.where(one > 0.0, one, 1.0)
        prob = jnp.where(one > 0.0, 1e-6 + num / safe, 1e-6 + one)
        dce = jnp.sum(-jnp.log(prob))
        pw = jnp.sum(pw_acc[...])
        out_ref[...] = jnp.reshape(dce + LAMBDA_ * pw, (1, 1))


def _assign(features, labels, prototypes, counts3, interpret=False):
    return pl.pallas_call(
        _assign_tc_kernel,
        out_shape=[
            jax.ShapeDtypeStruct((C, P, D), jnp.float32),
            jax.ShapeDtypeStruct((C, P, 1), jnp.float32),
        ],
        in_specs=[
            pl.BlockSpec(memory_space=pltpu.SMEM),
            pl.BlockSpec(memory_space=pltpu.VMEM),
            pl.BlockSpec(memory_space=pltpu.VMEM),
            pl.BlockSpec(memory_space=pltpu.VMEM),
        ],
        out_specs=[
            pl.BlockSpec(memory_space=pltpu.VMEM),
            pl.BlockSpec(memory_space=pltpu.VMEM),
        ],
        interpret=interpret,
    )(labels, features, prototypes, counts3)


def _loss(labels2d, features, protos, interpret=False):
    return pl.pallas_call(
        _loss_tc_kernel,
        grid=(C,),
        out_shape=jax.ShapeDtypeStruct((1, 1), jnp.float32),
        in_specs=[
            pl.BlockSpec((BATCH, 1), lambda c: (0, 0)),
            pl.BlockSpec((BATCH, D), lambda c: (0, 0)),
            pl.BlockSpec((1, P, D), lambda c: (c, 0, 0)),
        ],
        out_specs=pl.BlockSpec((1, 1), lambda c: (0, 0)),
        scratch_shapes=[
            pltpu.VMEM((BATCH, 1), jnp.float32),
            pltpu.VMEM((BATCH, 1), jnp.float32),
            pltpu.VMEM((BATCH, 1), jnp.float32),
        ],
        interpret=interpret,
    )(labels2d, features, protos)


def kernel(features, labels, prototypes, counts):
    labels = labels.astype(jnp.int32)
    s0 = _scores0(features, prototypes)
    protos_up = _assign_sc(features, labels, prototypes, counts, s0)
    out = _loss(labels[:, None], features, protos_up)
    return out[0, 0]


# speedup vs baseline: 10.0261x; 1.0062x over previous
import functools

import jax
import jax.numpy as jnp
from jax import lax
from jax.experimental import pallas as pl
from jax.experimental.pallas import tpu as pltpu
from jax.experimental.pallas import tpu_sc as plsc

GAMMA = 0.1
BPARAM = 10.0
TAO = 1.0
BETA = 1.0
LAMBDA_ = 0.1
EPS = 1e-6
C = 16
P = 512
D = 64
BATCH = 1024


def _assign_tc_kernel(labels_ref, feat_ref, protos_in, counts_in,
                      protos_out, counts_out):
    protos_out[...] = protos_in[...]
    counts_out[...] = counts_in[...]
    iota = jax.lax.broadcasted_iota(jnp.int32, (P, 1), 0)

    def body(i, carry):
        lab = labels_ref[i]
        frow = feat_ref[pl.ds(i, 1), :]
        cp = protos_out[lab]
        diff = frow - cp + EPS
        sq = jnp.sum(diff * diff, axis=1, keepdims=True)
        sq = jnp.maximum(sq, 1e-12)
        minval = jnp.min(sq)
        idx = jnp.min(jnp.where(sq == minval, iota, P))
        onehot = iota == idx
        cnt_col = counts_out[lab]
        cval = jnp.sum(jnp.where(onehot, cnt_col, 0.0))
        psel = jnp.sum(jnp.where(onehot, cp, 0.0), axis=0, keepdims=True)
        newp = (psel * cval + frow) / (cval + 1.0)
        protos_out[lab] = jnp.where(onehot, newp, cp)
        counts_out[lab] = cnt_col + jnp.where(onehot, 1.0, 0.0)
        return carry

    jax.lax.fori_loop(0, BATCH, body, 0)


L = 16
CP = C * P


def _scores0_tc_kernel(feat_ref, protos_ref, out_ref):
    pb = protos_ref[0]
    feats = feat_ref[...]
    dn = (((1,), (1,)), ((), ()))
    xy = jax.lax.dot_general(feats, pb, dn,
                             preferred_element_type=jnp.float32)
    ones_row = jnp.ones((1, D), jnp.float32)
    ynrow = jax.lax.dot_general(ones_row, pb * pb, dn,
                                preferred_element_type=jnp.float32)
    ysrow = jax.lax.dot_general(ones_row, pb, dn,
                                preferred_element_type=jnp.float32)
    out_ref[...] = (ynrow - 2.0 * xy - 2.0 * EPS * ysrow)[None, :, :]


def _scores0(features, protos):
    return pl.pallas_call(
        _scores0_tc_kernel,
        grid=(C,),
        out_shape=jax.ShapeDtypeStruct((C, BATCH, P), jnp.float32),
        in_specs=[
            pl.BlockSpec((BATCH, D), lambda c: (0, 0)),
            pl.BlockSpec((1, P, D), lambda c: (c, 0, 0)),
        ],
        out_specs=pl.BlockSpec((1, BATCH, P), lambda c: (c, 0, 0)),
    )(features, protos)


def _assign_sc(feat_flat, labels, protos_flat, counts, scores0_flat):
    mesh = plsc.VectorSubcoreMesh(core_axis_name="c", subcore_axis_name="s")

    @functools.partial(
        pl.kernel,
        mesh=mesh,
        out_type=jax.ShapeDtypeStruct((C, P, D), jnp.float32),
        compiler_params=pltpu.CompilerParams(needs_layout_passes=False),
        scratch_types=[
            pltpu.VMEM((BATCH,), jnp.int32),
            pltpu.VMEM((BATCH,), jnp.int32),
            pltpu.VMEM((P, D), jnp.float32),
            pltpu.VMEM((P * D,), jnp.float32),
            pltpu.VMEM((P,), jnp.float32),
            pltpu.VMEM((2, P), jnp.float32),
            pltpu.VMEM((2, D), jnp.float32),
            pltpu.VMEM((P,), jnp.float32),
            pltpu.VMEM((P,), jnp.int32),
            pltpu.VMEM((P,), jnp.int32),
            pltpu.SemaphoreType.DMA,
            pltpu.SemaphoreType.DMA,
        ],
    )
    def k(feat_hbm, labels_hbm, protos_hbm, counts_hbm, scores0_hbm, out_hbm,
          labels_v, myidx_v, protos_v, protosT_v, counts_v,
          scorebuf, frow, ynys_v, dlist, dflag, sem_s, sem_f):
        cid = lax.axis_index("c")
        sid = lax.axis_index("s")
        cls = sid
        lane = lax.iota(jnp.int32, L)

        @pl.when(cid == 0)
        def _body():
            pltpu.sync_copy(labels_hbm, labels_v)
            pltpu.sync_copy(protos_hbm.at[cls], protos_v)
            pltpu.sync_copy(counts_hbm.at[cls], counts_v)

            ones_i = jnp.full((L,), 1, jnp.int32)
            zeros_i = jnp.full((L,), 0, jnp.int32)
            zeros_f = jnp.zeros((L,), jnp.float32)

            def cbody(t, nacc):
                lv = labels_v[pl.ds(t * L, L)]
                mask = lv == cls
                cntv = plsc.all_reduce_population_count(mask)
                keys = jnp.where(mask, zeros_i, ones_i)
                _, sorted_idx = lax.sort((keys, lane + t * L), num_keys=1)
                plsc.store_scatter(myidx_v, [nacc + lane], sorted_idx,
                                   mask=lane < cntv)
                return nacc + cntv[0]

            n = lax.fori_loop(0, BATCH // L, cbody, 0)

            def ibody(t, _):
                dflag[pl.ds(t * L, L)] = zeros_i
                dlist[pl.ds(t * L, L)] = zeros_i
                return 0

            lax.fori_loop(0, P // L, ibody, 0)

            big = jnp.float32(3.0e38)

            @pl.when(n > 0)
            def _chain():
                smp0 = plsc.load_gather(myidx_v, [zeros_i])[0]
                pltpu.async_copy(scores0_hbm.at[cls, smp0],
                                 scorebuf.at[0], sem_s)
                pltpu.async_copy(feat_hbm.at[smp0], frow.at[0], sem_f)

                def sample_body(i, ndirty):
                    par = lax.rem(i, 2)
                    pltpu.make_async_copy(scores0_hbm.at[0, 0],
                                          scorebuf.at[par], sem_s).wait()
                    pltpu.make_async_copy(feat_hbm.at[0],
                                          frow.at[par], sem_f).wait()

                    @pl.when(i + 1 < n)
                    def _pf():
                        nx = plsc.load_gather(
                            myidx_v, [jnp.full((L,), i + 1, jnp.int32)])[0]
                        npar = lax.rem(i + 1, 2)
                        pltpu.async_copy(scores0_hbm.at[cls, nx],
                                         scorebuf.at[npar], sem_s)
                        pltpu.async_copy(feat_hbm.at[nx], frow.at[npar],
                                         sem_f)

                    fr = [frow[par, pl.ds(q * L, L)] for q in range(D // L)]

                    def corr(t, _):
                        jvec = dlist[pl.ds(t * L, L)]
                        valid = (t * L + lane) < ndirty
                        ynysv = plsc.load_gather(ynys_v, [jvec])
                        acc = zeros_f
                        for d in range(D):
                            pd = plsc.load_gather(protosT_v, [d * P + jvec])
                            acc = acc + fr[d // L][d % L] * pd
                        plsc.store_scatter(
                            scorebuf, [jnp.full((L,), par, jnp.int32), jvec],
                            ynysv - 2.0 * acc, mask=valid)
                        return 0

                    lax.fori_loop(0, (ndirty + L - 1) // L, corr, 0)

                    def grp(g4, carry):
                        mv, mi = carry
                        for u in range(4):
                            g = g4 * 4 + u
                            sv = scorebuf[par, pl.ds(g * L, L)]
                            jidx = g * L + lane
                            lt = sv < mv
                            mv = jnp.where(lt, sv, mv)
                            mi = jnp.where(lt, jidx, mi)
                        return (mv, mi)

                    minval, minidx = lax.fori_loop(
                        0, P // (L * 4), grp,
                        (jnp.full((L,), big), jnp.zeros((L,), jnp.int32)))
                    gmin = lax.sort(minval)[0]
                    cand = jnp.where(minval == gmin, minidx,
                                     jnp.full((L,), P, jnp.int32))
                    j = lax.sort(cand)[0]

                    jsplat = jnp.full((L,), j, jnp.int32)
                    cvec = plsc.load_gather(counts_v, [jsplat])
                    newcv = cvec + 1.0
                    accy = zeros_f
                    for q in range(D // L):
                        pq = protos_v[j, pl.ds(q * L, L)]
                        npq = (pq * cvec + fr[q]) / newcv
                        protos_v[j, pl.ds(q * L, L)] = npq
                        plsc.store_scatter(protosT_v,
                                           [(q * L + lane) * P + j], npq)
                        accy = accy + npq * npq - (2.0 * EPS) * npq
                    s = accy[0]
                    for r in range(1, L):
                        s = s + accy[r]
                    plsc.store_scatter(ynys_v, [jsplat],
                                       jnp.full((L,), s, jnp.float32),
                                       mask=lane == 0)
                    plsc.store_scatter(counts_v, [jsplat], newcv,
                                       mask=lane == 0)

                    flagv = plsc.load_gather(dflag, [jsplat])
                    newmask = (lane == 0) & (flagv == 0)
                    plsc.store_scatter(
                        dlist, [jnp.full((L,), ndirty, jnp.int32)], jsplat,
                        mask=newmask)
                    plsc.store_scatter(dflag, [jsplat], ones_i, mask=newmask)
                    return ndirty + 1 - flagv[0]

                lax.fori_loop(0, n, sample_body, 0)

            pltpu.sync_copy(protos_v, out_hbm.at[cls])

    return k(feat_flat, labels, protos_flat, counts, scores0_flat)


def _loss_tc_kernel(labels_ref, feat_ref, protos_ref,
                    out_ref, one_acc, num_acc, pw_acc):
    c = pl.program_id(0)

    @pl.when(c == 0)
    def _init():
        one_acc[...] = jnp.zeros_like(one_acc)
        num_acc[...] = jnp.zeros_like(num_acc)
        pw_acc[...] = jnp.zeros_like(pw_acc)

    pb = protos_ref[0]
    feats = feat_ref[...]
    dn = (((1,), (1,)), ((), ()))
    xy = jax.lax.dot_general(feats, pb, dn,
                             preferred_element_type=jnp.float32)
    ones_row = jnp.ones((1, D), jnp.float32)
    ynrow = jax.lax.dot_general(ones_row, pb * pb, dn,
                                preferred_element_type=jnp.float32)
    ysrow = jax.lax.dot_general(ones_row, pb, dn,
                                preferred_element_type=jnp.float32)
    xn = jnp.sum(feats * feats, axis=1, keepdims=True)
    xs = jnp.sum(feats, axis=1, keepdims=True)
    xrow = xn + 2.0 * EPS * xs + D * EPS * EPS
    yrow = ynrow - 2.0 * EPS * ysrow
    sq = (xrow + yrow) - 2.0 * xy
    sq = jnp.maximum(sq, 1e-12)
    expterm = jnp.exp(-GAMMA * sq)
    ones_col = jnp.ones((P, 1), jnp.float32)
    pc = jax.lax.dot_general(expterm, ones_col,
                             (((1,), (0,)), ((), ())),
                             preferred_element_type=jnp.float32)
    lab = labels_ref[...]
    mask = lab == c
    one_acc[...] += pc
    num_acc[...] += jnp.where(mask, pc, 0.0)
    dmin = jnp.sqrt(jnp.min(sq, axis=1, keepdims=True))
    sign = jnp.where(mask, 1.0, -1.0)
    z = BPARAM - (TAO - dmin) * sign
    soft = jnp.log(1.0 + jnp.exp(BETA * jnp.minimum(z, 10.0))) / BETA
    pw_acc[...] += jnp.where(z > 10.0, z, soft)

    @pl.when(c == C - 1)
    def _fin():
        one = one_acc[...]
        num = num_acc[...]
        safe = jnp.where(one > 0.0, one, 1.0)
        prob = jnp.where(one > 0.0, 1e-6 + num / safe, 1e-6 + one)
        dce = jnp.sum(-jnp.log(prob))
        pw = jnp.sum(pw_acc[...])
        out_ref[...] = jnp.reshape(dce + LAMBDA_ * pw, (1, 1))


def _assign(features, labels, prototypes, counts3, interpret=False):
    return pl.pallas_call(
        _assign_tc_kernel,
        out_shape=[
            jax.ShapeDtypeStruct((C, P, D), jnp.float32),
            jax.ShapeDtypeStruct((C, P, 1), jnp.float32),
        ],
        in_specs=[
            pl.BlockSpec(memory_space=pltpu.SMEM),
            pl.BlockSpec(memory_space=pltpu.VMEM),
            pl.BlockSpec(memory_space=pltpu.VMEM),
            pl.BlockSpec(memory_space=pltpu.VMEM),
        ],
        out_specs=[
            pl.BlockSpec(memory_space=pltpu.VMEM),
            pl.BlockSpec(memory_space=pltpu.VMEM),
        ],
        interpret=interpret,
    )(labels, features, prototypes, counts3)


def _loss(labels2d, features, protos, interpret=False):
    return pl.pallas_call(
        _loss_tc_kernel,
        grid=(C,),
        out_shape=jax.ShapeDtypeStruct((1, 1), jnp.float32),
        in_specs=[
            pl.BlockSpec((BATCH, 1), lambda c: (0, 0)),
            pl.BlockSpec((BATCH, D), lambda c: (0, 0)),
            pl.BlockSpec((1, P, D), lambda c: (c, 0, 0)),
        ],
        out_specs=pl.BlockSpec((1, 1), lambda c: (0, 0)),
        scratch_shapes=[
            pltpu.VMEM((BATCH, 1), jnp.float32),
            pltpu.VMEM((BATCH, 1), jnp.float32),
            pltpu.VMEM((BATCH, 1), jnp.float32),
        ],
        interpret=interpret,
    )(labels2d, features, protos)


def kernel(features, labels, prototypes, counts):
    labels = labels.astype(jnp.int32)
    s0 = _scores0(features, prototypes)
    protos_up = _assign_sc(features, labels, prototypes, counts, s0)
    out = _loss(labels[:, None], features, protos_up)
    return out[0, 0]
